# baseline (speedup 1.0000x reference)
import jax
import jax.numpy as jnp
from jax.experimental import pallas as pl
from jax.experimental.pallas import tpu as pltpu

_K = 256
_ROWS = 64
_COLS = 8192


def _sparsify_kernel(x_ref, o_ref):
    x = x_ref[...]
    i = jax.lax.bitcast_convert_type(x, jnp.int32)
    keys = jnp.where(i >= 0, i, i ^ jnp.int32(0x7FFFFFFF))
    int_min = jnp.int32(-2147483648)
    biased = keys ^ int_min
    hi_u = jax.lax.shift_right_logical(biased, 16)
    lo_u = biased & jnp.int32(0xFFFF)
    hi16 = (hi_u - 32768).astype(jnp.int16)
    lo16 = (lo_u - 32768).astype(jnp.int16)

    def search16(arr16):
        def body(j, tb):
            trial = tb | (jnp.int32(1) << (jnp.int32(15) - j))
            th = (trial - 32768).astype(jnp.int16)
            t = (arr16 >= th).astype(jnp.int16)
            w = _COLS
            while w > 128:
                w //= 2
                t = t[:, :w] + t[:, w:]
            cnt = jnp.sum(t.astype(jnp.int32), axis=1, keepdims=True)
            return jnp.where(cnt >= _K, trial, tb)

        tb0 = jnp.zeros((arr16.shape[0], 1), jnp.int32)
        return jax.lax.fori_loop(0, 16, body, tb0)

    h_u = search16(hi16)

    z_u = jnp.where(hi_u > h_u, jnp.int32(0xFFFF),
                    jnp.where(hi_u == h_u, lo_u, jnp.int32(0)))
    z16 = (z_u - 32768).astype(jnp.int16)

    l_u = search16(z16)

    tkey = ((h_u << 16) | l_u) ^ int_min
    o_ref[...] = jnp.where(keys >= tkey, x, jnp.float32(0.0))


def kernel(x):
    return pl.pallas_call(
        _sparsify_kernel,
        grid=(1,),
        in_specs=[pl.BlockSpec((_ROWS, _COLS), lambda i: (0, 0))],
        out_specs=pl.BlockSpec((_ROWS, _COLS), lambda i: (0, 0)),
        out_shape=jax.ShapeDtypeStruct((_ROWS, _COLS), jnp.float32),
    )(x)


# speedup vs baseline: 1.0449x; 1.0449x over previous
import jax
import jax.numpy as jnp
from jax.experimental import pallas as pl
from jax.experimental.pallas import tpu as pltpu

_K = 256
_ROWS = 64
_COLS = 8192
_BLOCK_ROWS = 64


def _sparsify_kernel(x_ref, o_ref):
    x = x_ref[...]
    i = jax.lax.bitcast_convert_type(x, jnp.int32)
    keys = jnp.where(i >= 0, i, i ^ jnp.int32(0x7FFFFFFF))
    int_min = jnp.int32(-2147483648)

    def body(j, tb):
        bit = jnp.int32(1) << (jnp.int32(31) - j)
        trial = tb | bit
        thresh = trial ^ int_min
        t = jnp.where(keys >= thresh, jnp.float32(1), jnp.float32(0))
        cnt = jnp.sum(t, axis=1, keepdims=True)
        return jnp.where(cnt >= jnp.float32(_K), trial, tb)

    tb0 = jnp.zeros((x.shape[0], 1), jnp.int32)
    tb = jax.lax.fori_loop(0, 32, body, tb0)
    tkey = tb ^ int_min
    o_ref[...] = jnp.where(keys >= tkey, x, jnp.float32(0.0))


def kernel(x):
    grid = (_ROWS // _BLOCK_ROWS,)
    return pl.pallas_call(
        _sparsify_kernel,
        grid=grid,
        in_specs=[pl.BlockSpec((_BLOCK_ROWS, _COLS), lambda i: (i, 0))],
        out_specs=pl.BlockSpec((_BLOCK_ROWS, _COLS), lambda i: (i, 0)),
        out_shape=jax.ShapeDtypeStruct((_ROWS, _COLS), jnp.float32),
    )(x)
